# baseline (speedup 1.0000x reference)
import jax
import jax.numpy as jnp
from jax import lax
from jax.experimental import pallas as pl
from jax.experimental.pallas import tpu as pltpu
from jax.experimental.pallas import tpu_sc as plsc

N = 10000
E = 320000
D = 128
WIN = 128
NC = 2
NS = 16
WPT = 80
CW = 16
NCHUNK = 5
NWINP = NC * NS * WPT
EPAD = NWINP * WIN - E
ACC_ROWS = 10240
RPT = ACC_ROWS // NS
NDUP = 2

_mesh = plsc.VectorSubcoreMesh(core_axis_name="c", subcore_axis_name="s")


def _zero_fill(buf, rows, width):
    zero = jnp.zeros((16,), jnp.float32)

    def row(i, _):
        for j in range(width // 16):
            buf[i, pl.ds(j * 16, 16)] = zero
        return 0

    lax.fori_loop(0, rows, row, 0)


def _zero_acc(acc, zbuf, s, width):
    _zero_fill(zbuf, 128, width)
    for k in range(RPT // 128):
        pltpu.sync_copy(zbuf, acc.at[pl.ds(s * RPT + k * 128, 128)])


def _writeback(acc, outp, c, s):
    for k in range(RPT // 128):
        pltpu.sync_copy(acc.at[pl.ds(s * RPT + k * 128, 128)],
                        outp.at[c, pl.ds(s * RPT + k * 128, 128)])


def _deg_body(dst2d, outp, acc, idxbuf, onesbuf):
    c = lax.axis_index("c")
    s = lax.axis_index("s")
    base = (c * NS + s) * WPT

    _zero_acc(acc, onesbuf, s, 16)
    pltpu.sync_copy(dst2d.at[pl.ds(base, WPT)], idxbuf)

    one = jnp.ones((16,), jnp.float32)

    def fill(i, _):
        onesbuf[i, :] = one
        return 0

    lax.fori_loop(0, 128, fill, 0)
    plsc.subcore_barrier()

    def win(j, _):
        pltpu.sync_copy(onesbuf, acc.at[idxbuf.at[j]], add=True)
        return 0

    lax.fori_loop(0, WPT, win, 0)
    plsc.subcore_barrier()
    _writeback(acc, outp, c, s)


_deg_call = pl.kernel(
    _deg_body,
    out_type=jax.ShapeDtypeStruct((NC, ACC_ROWS, 16), jnp.float32),
    mesh=_mesh,
    scratch_types=[
        pltpu.VMEM_SHARED((ACC_ROWS, 16), jnp.float32),
        pltpu.VMEM((WPT, WIN), jnp.int32),
        pltpu.VMEM((128, 16), jnp.float32),
    ],
)


def _spmm_body(xp, src2d, dst2d, outp, acc, sbuf, dbuf, rbuf0, rbuf1,
               gsem0, gsem1, ssem0, ssem1, isem):
    c = lax.axis_index("c")
    s = lax.axis_index("s")
    base = (c * NS + s) * WPT

    _zero_acc(acc, rbuf0, s, D)
    plsc.subcore_barrier()

    pltpu.sync_copy(src2d.at[pl.ds(base, CW)], sbuf.at[0])
    pltpu.sync_copy(dst2d.at[pl.ds(base, CW)], dbuf.at[0])
    pltpu.async_copy(xp.at[sbuf.at[0, 0]], rbuf0, gsem0)
    pltpu.async_copy(xp.at[sbuf.at[0, 1]], rbuf1, gsem1)

    def chunk(ch, _):
        par = ch % 2
        nxt = 1 - par

        @pl.when(ch + 1 < NCHUNK)
        def _():
            pltpu.async_copy(src2d.at[pl.ds(base + (ch + 1) * CW, CW)],
                             sbuf.at[nxt], isem)
            pltpu.async_copy(dst2d.at[pl.ds(base + (ch + 1) * CW, CW)],
                             dbuf.at[nxt], isem)

        def step(t, _):
            j0 = 2 * t
            j1 = j0 + 1
            pltpu.make_async_copy(xp.at[sbuf.at[par, j0]], rbuf0, gsem0).wait()
            pltpu.async_copy(rbuf0, acc.at[dbuf.at[par, j0]], ssem0, add=True)
            pltpu.make_async_copy(xp.at[sbuf.at[par, j1]], rbuf1, gsem1).wait()
            pltpu.async_copy(rbuf1, acc.at[dbuf.at[par, j1]], ssem1, add=True)

            @pl.when(j0 + 2 < CW)
            def _():
                pltpu.make_async_copy(
                    rbuf0, acc.at[dbuf.at[par, j0]], ssem0).wait()
                pltpu.async_copy(xp.at[sbuf.at[par, j0 + 2]], rbuf0, gsem0)
                pltpu.make_async_copy(
                    rbuf1, acc.at[dbuf.at[par, j1]], ssem1).wait()
                pltpu.async_copy(xp.at[sbuf.at[par, j1 + 2]], rbuf1, gsem1)

            return 0

        lax.fori_loop(0, CW // 2, step, 0)
        pltpu.make_async_copy(rbuf0, acc.at[dbuf.at[par, CW - 2]], ssem0).wait()
        pltpu.make_async_copy(rbuf1, acc.at[dbuf.at[par, CW - 1]], ssem1).wait()

        @pl.when(ch + 1 < NCHUNK)
        def _():
            pltpu.make_async_copy(src2d.at[pl.ds(base + (ch + 1) * CW, CW)],
                                  sbuf.at[nxt], isem).wait()
            pltpu.make_async_copy(dst2d.at[pl.ds(base + (ch + 1) * CW, CW)],
                                  dbuf.at[nxt], isem).wait()
            pltpu.async_copy(xp.at[sbuf.at[nxt, 0]], rbuf0, gsem0)
            pltpu.async_copy(xp.at[sbuf.at[nxt, 1]], rbuf1, gsem1)

        return 0

    lax.fori_loop(0, NCHUNK, chunk, 0)
    plsc.subcore_barrier()
    _writeback(acc, outp, c, s)


_spmm_call = pl.kernel(
    _spmm_body,
    out_type=jax.ShapeDtypeStruct((NC, ACC_ROWS, D), jnp.float32),
    mesh=_mesh,
    scratch_types=[
        pltpu.VMEM_SHARED((ACC_ROWS, D), jnp.float32),
        pltpu.VMEM((2, CW, WIN), jnp.int32),
        pltpu.VMEM((2, CW, WIN), jnp.int32),
        pltpu.VMEM((WIN, D), jnp.float32),
        pltpu.VMEM((WIN, D), jnp.float32),
        pltpu.SemaphoreType.DMA,
        pltpu.SemaphoreType.DMA,
        pltpu.SemaphoreType.DMA,
        pltpu.SemaphoreType.DMA,
        pltpu.SemaphoreType.DMA,
    ],
)


BR = 2000
_GRID = (N // BR,)


def _dinv_of(degp):
    return lax.rsqrt(degp[0, :, 0:1] + degp[1, :, 0:1] + 1.0)


def _scale_body(degp, x, o):
    o[...] = x[...] * _dinv_of(degp)


def _layer1_body(degp, s0p, xp, w1, b1, o):
    dinv = _dinv_of(degp)
    p = (s0p[0] + s0p[1] + xp[...]) * dinv
    h = jnp.dot(p, w1[...], preferred_element_type=jnp.float32) + b1[...]
    o[...] = jnp.maximum(h, 0.0) * dinv


def _layer2_body(degp, s1p, h1p, w2, b2, wcat, o):
    dinv = _dinv_of(degp)
    p = (s1p[0] + s1p[1] + h1p[...]) * dinv
    h2 = jnp.dot(p, w2[...], preferred_element_type=jnp.float32) + b2[...]
    h2 = jnp.maximum(h2, 0.0)
    o[...] = jnp.dot(h2, wcat[...], preferred_element_type=jnp.float32) * dinv


def _out_body(degp, s2p, gp, bmu, bls, mu, ls):
    dinv = _dinv_of(degp)
    p = (s2p[0] + s2p[1] + gp[...]) * dinv
    mu[...] = p[:, :64] + bmu[...]
    ls[...] = p[:, 64:] + bls[...]


def _rows(block_w):
    return pl.BlockSpec((BR, block_w), lambda i: (i, 0))


def _parts(block_w):
    return pl.BlockSpec((NC, BR, block_w), lambda i: (0, i, 0))


def _full(shape):
    nd = len(shape)
    return pl.BlockSpec(shape, lambda i: (0,) * nd)


_degp_spec = _parts(16)

_scale = pl.pallas_call(
    _scale_body,
    grid=_GRID,
    in_specs=[_degp_spec, _rows(D)],
    out_specs=_rows(D),
    out_shape=jax.ShapeDtypeStruct((N, D), jnp.float32),
)

_layer1 = pl.pallas_call(
    _layer1_body,
    grid=_GRID,
    in_specs=[_degp_spec, _parts(D), _rows(D), _full((D, D)), _full((1, D))],
    out_specs=_rows(D),
    out_shape=jax.ShapeDtypeStruct((N, D), jnp.float32),
)

_layer2 = pl.pallas_call(
    _layer2_body,
    grid=_GRID,
    in_specs=[_degp_spec, _parts(D), _rows(D), _full((D, 256)),
              _full((1, 256)), _full((256, D))],
    out_specs=_rows(D),
    out_shape=jax.ShapeDtypeStruct((N, D), jnp.float32),
)

_outs = pl.pallas_call(
    _out_body,
    grid=_GRID,
    in_specs=[_degp_spec, _parts(D), _rows(D), _full((1, 64)), _full((1, 64))],
    out_specs=[_rows(64), _rows(64)],
    out_shape=[jax.ShapeDtypeStruct((N, 64), jnp.float32),
               jax.ShapeDtypeStruct((N, 64), jnp.float32)],
)


def kernel(x, edge_index, W1, b1, W2, b2, Wmu, bmu, Wls, bls):
    ar = jnp.arange(EPAD, dtype=jnp.int32)
    pad_src = ar % 4096
    pad_dst = N + (ar % (ACC_ROWS - N))
    dup = (jnp.arange(E, dtype=jnp.int32) % NDUP) * N
    src2d = jnp.concatenate([edge_index[0] + dup, pad_src]).reshape(NWINP, WIN)
    dst2d = jnp.concatenate([edge_index[1], pad_dst]).reshape(NWINP, WIN)
    wcat = jnp.concatenate([Wmu, Wls], axis=1)
    b1r = b1.reshape(1, D)
    b2r = b2.reshape(1, 256)
    bmur = bmu.reshape(1, 64)
    blsr = bls.reshape(1, 64)

    degp = _deg_call(dst2d)
    xp = _scale(degp, x)
    s0p = _spmm_call(jnp.concatenate([xp] * NDUP), src2d, dst2d)
    h1p = _layer1(degp, s0p, xp, W1, b1r)
    s1p = _spmm_call(jnp.concatenate([h1p] * NDUP), src2d, dst2d)
    gp = _layer2(degp, s1p, h1p, W2, b2r, wcat)
    s2p = _spmm_call(jnp.concatenate([gp] * NDUP), src2d, dst2d)
    mu, ls = _outs(degp, s2p, gp, bmur, blsr)
    return (mu, ls)


# speedup vs baseline: 1.0641x; 1.0641x over previous
import jax
import jax.numpy as jnp
from jax import lax
from jax.experimental import pallas as pl
from jax.experimental.pallas import tpu as pltpu
from jax.experimental.pallas import tpu_sc as plsc

N = 10000
E = 320000
D = 128
WIN = 128
NC = 2
NS = 16
WPT = 80
CW = 16
NCHUNK = 5
NWINP = NC * NS * WPT
EPAD = NWINP * WIN - E
ACC_ROWS = 10240
RPT = ACC_ROWS // NS

_mesh = plsc.VectorSubcoreMesh(core_axis_name="c", subcore_axis_name="s")


def _zero_fill(buf, rows, width):
    zero = jnp.zeros((16,), jnp.float32)

    def row(i, _):
        for j in range(width // 16):
            buf[i, pl.ds(j * 16, 16)] = zero
        return 0

    lax.fori_loop(0, rows, row, 0)


def _zero_acc(acc, zbuf, s, width):
    _zero_fill(zbuf, 128, width)
    for k in range(RPT // 128):
        pltpu.sync_copy(zbuf, acc.at[pl.ds(s * RPT + k * 128, 128)])


def _writeback(acc, outp, c, s):
    for k in range(RPT // 128):
        pltpu.sync_copy(acc.at[pl.ds(s * RPT + k * 128, 128)],
                        outp.at[c, pl.ds(s * RPT + k * 128, 128)])


def _deg_body(dst2d, outp, acc, idxbuf, onesbuf, dsem):
    c = lax.axis_index("c")
    s = lax.axis_index("s")
    base = (c * NS + s) * WPT

    _zero_acc(acc, onesbuf, s, 16)
    pltpu.sync_copy(dst2d.at[pl.ds(base, WPT)], idxbuf)

    one = jnp.ones((16,), jnp.float32)

    def fill(i, _):
        onesbuf[i, :] = one
        return 0

    lax.fori_loop(0, 128, fill, 0)
    plsc.subcore_barrier()

    def win(j, _):
        pltpu.async_copy(onesbuf, acc.at[idxbuf.at[j]], dsem, add=True)

        @pl.when(j >= 8)
        def _():
            pltpu.make_async_copy(onesbuf, acc.at[idxbuf.at[j]], dsem).wait()

        return 0

    lax.fori_loop(0, WPT, win, 0)
    for _ in range(8):
        pltpu.make_async_copy(onesbuf, acc.at[idxbuf.at[0]], dsem).wait()
    plsc.subcore_barrier()
    _writeback(acc, outp, c, s)


_deg_call = pl.kernel(
    _deg_body,
    out_type=jax.ShapeDtypeStruct((NC, ACC_ROWS, 16), jnp.float32),
    mesh=_mesh,
    scratch_types=[
        pltpu.VMEM_SHARED((ACC_ROWS, 16), jnp.float32),
        pltpu.VMEM((WPT, WIN), jnp.int32),
        pltpu.VMEM((128, 16), jnp.float32),
        pltpu.SemaphoreType.DMA,
    ],
)


def _spmm_body(xp, src2d, dst2d, outp, acc, sbuf, dbuf, rbuf0, rbuf1,
               gsem0, gsem1, ssem0, ssem1, isem):
    c = lax.axis_index("c")
    s = lax.axis_index("s")
    base = (c * NS + s) * WPT

    _zero_acc(acc, rbuf0, s, D)
    plsc.subcore_barrier()

    pltpu.sync_copy(src2d.at[pl.ds(base, CW)], sbuf.at[0])
    pltpu.sync_copy(dst2d.at[pl.ds(base, CW)], dbuf.at[0])
    pltpu.async_copy(xp.at[sbuf.at[0, 0]], rbuf0, gsem0)
    pltpu.async_copy(xp.at[sbuf.at[0, 1]], rbuf1, gsem1)

    def chunk(ch, _):
        par = ch % 2
        nxt = 1 - par

        @pl.when(ch + 1 < NCHUNK)
        def _():
            pltpu.async_copy(src2d.at[pl.ds(base + (ch + 1) * CW, CW)],
                             sbuf.at[nxt], isem)
            pltpu.async_copy(dst2d.at[pl.ds(base + (ch + 1) * CW, CW)],
                             dbuf.at[nxt], isem)

        def step(t, _):
            j0 = 2 * t
            j1 = j0 + 1
            pltpu.make_async_copy(xp.at[sbuf.at[par, j0]], rbuf0, gsem0).wait()
            pltpu.async_copy(rbuf0, acc.at[dbuf.at[par, j0]], ssem0, add=True)
            pltpu.make_async_copy(xp.at[sbuf.at[par, j1]], rbuf1, gsem1).wait()
            pltpu.async_copy(rbuf1, acc.at[dbuf.at[par, j1]], ssem1, add=True)

            @pl.when(j0 + 2 < CW)
            def _():
                pltpu.make_async_copy(
                    rbuf0, acc.at[dbuf.at[par, j0]], ssem0).wait()
                pltpu.async_copy(xp.at[sbuf.at[par, j0 + 2]], rbuf0, gsem0)
                pltpu.make_async_copy(
                    rbuf1, acc.at[dbuf.at[par, j1]], ssem1).wait()
                pltpu.async_copy(xp.at[sbuf.at[par, j1 + 2]], rbuf1, gsem1)

            return 0

        lax.fori_loop(0, CW // 2, step, 0)
        pltpu.make_async_copy(rbuf0, acc.at[dbuf.at[par, CW - 2]], ssem0).wait()
        pltpu.make_async_copy(rbuf1, acc.at[dbuf.at[par, CW - 1]], ssem1).wait()

        @pl.when(ch + 1 < NCHUNK)
        def _():
            pltpu.make_async_copy(src2d.at[pl.ds(base + (ch + 1) * CW, CW)],
                                  sbuf.at[nxt], isem).wait()
            pltpu.make_async_copy(dst2d.at[pl.ds(base + (ch + 1) * CW, CW)],
                                  dbuf.at[nxt], isem).wait()
            pltpu.async_copy(xp.at[sbuf.at[nxt, 0]], rbuf0, gsem0)
            pltpu.async_copy(xp.at[sbuf.at[nxt, 1]], rbuf1, gsem1)

        return 0

    lax.fori_loop(0, NCHUNK, chunk, 0)
    plsc.subcore_barrier()
    _writeback(acc, outp, c, s)


_spmm_call = pl.kernel(
    _spmm_body,
    out_type=jax.ShapeDtypeStruct((NC, ACC_ROWS, D), jnp.float32),
    mesh=_mesh,
    scratch_types=[
        pltpu.VMEM_SHARED((ACC_ROWS, D), jnp.float32),
        pltpu.VMEM((2, CW, WIN), jnp.int32),
        pltpu.VMEM((2, CW, WIN), jnp.int32),
        pltpu.VMEM((WIN, D), jnp.float32),
        pltpu.VMEM((WIN, D), jnp.float32),
        pltpu.SemaphoreType.DMA,
        pltpu.SemaphoreType.DMA,
        pltpu.SemaphoreType.DMA,
        pltpu.SemaphoreType.DMA,
        pltpu.SemaphoreType.DMA,
    ],
)


BR = 2000
_GRID = (N // BR,)


def _dinv_of(degp):
    return lax.rsqrt(degp[0, :, 0:1] + degp[1, :, 0:1] + 1.0)


def _scale_body(degp, x, o):
    o[...] = x[...] * _dinv_of(degp)


def _layer1_body(degp, s0p, xp, w1, b1, o):
    dinv = _dinv_of(degp)
    p = (s0p[0] + s0p[1] + xp[...]) * dinv
    h = jnp.dot(p, w1[...], preferred_element_type=jnp.float32) + b1[...]
    o[...] = jnp.maximum(h, 0.0) * dinv


def _layer2_body(degp, s1p, h1p, w2, b2, wcat, o):
    dinv = _dinv_of(degp)
    p = (s1p[0] + s1p[1] + h1p[...]) * dinv
    h2 = jnp.dot(p, w2[...], preferred_element_type=jnp.float32) + b2[...]
    h2 = jnp.maximum(h2, 0.0)
    o[...] = jnp.dot(h2, wcat[...], preferred_element_type=jnp.float32) * dinv


def _out_body(degp, s2p, gp, bmu, bls, mu, ls):
    dinv = _dinv_of(degp)
    p = (s2p[0] + s2p[1] + gp[...]) * dinv
    mu[...] = p[:, :64] + bmu[...]
    ls[...] = p[:, 64:] + bls[...]


def _rows(block_w):
    return pl.BlockSpec((BR, block_w), lambda i: (i, 0))


def _parts(block_w):
    return pl.BlockSpec((NC, BR, block_w), lambda i: (0, i, 0))


def _full(shape):
    nd = len(shape)
    return pl.BlockSpec(shape, lambda i: (0,) * nd)


_degp_spec = _parts(16)

_scale = pl.pallas_call(
    _scale_body,
    grid=_GRID,
    in_specs=[_degp_spec, _rows(D)],
    out_specs=_rows(D),
    out_shape=jax.ShapeDtypeStruct((N, D), jnp.float32),
)

_layer1 = pl.pallas_call(
    _layer1_body,
    grid=_GRID,
    in_specs=[_degp_spec, _parts(D), _rows(D), _full((D, D)), _full((1, D))],
    out_specs=_rows(D),
    out_shape=jax.ShapeDtypeStruct((N, D), jnp.float32),
)

_layer2 = pl.pallas_call(
    _layer2_body,
    grid=_GRID,
    in_specs=[_degp_spec, _parts(D), _rows(D), _full((D, 256)),
              _full((1, 256)), _full((256, D))],
    out_specs=_rows(D),
    out_shape=jax.ShapeDtypeStruct((N, D), jnp.float32),
)

_outs = pl.pallas_call(
    _out_body,
    grid=_GRID,
    in_specs=[_degp_spec, _parts(D), _rows(D), _full((1, 64)), _full((1, 64))],
    out_specs=[_rows(64), _rows(64)],
    out_shape=[jax.ShapeDtypeStruct((N, 64), jnp.float32),
               jax.ShapeDtypeStruct((N, 64), jnp.float32)],
)


def kernel(x, edge_index, W1, b1, W2, b2, Wmu, bmu, Wls, bls):
    ar = jnp.arange(EPAD, dtype=jnp.int32)
    pad_src = ar % 4096
    pad_dst = N + (ar % (ACC_ROWS - N))
    src2d = jnp.concatenate([edge_index[0], pad_src]).reshape(NWINP, WIN)
    dst2d = jnp.concatenate([edge_index[1], pad_dst]).reshape(NWINP, WIN)
    wcat = jnp.concatenate([Wmu, Wls], axis=1)
    b1r = b1.reshape(1, D)
    b2r = b2.reshape(1, 256)
    bmur = bmu.reshape(1, 64)
    blsr = bls.reshape(1, 64)

    degp = _deg_call(dst2d)
    xp = _scale(degp, x)
    s0p = _spmm_call(xp, src2d, dst2d)
    h1p = _layer1(degp, s0p, xp, W1, b1r)
    s1p = _spmm_call(h1p, src2d, dst2d)
    gp = _layer2(degp, s1p, h1p, W2, b2r, wcat)
    s2p = _spmm_call(gp, src2d, dst2d)
    mu, ls = _outs(degp, s2p, gp, bmur, blsr)
    return (mu, ls)


# speedup vs baseline: 1.1512x; 1.0818x over previous
import jax
import jax.numpy as jnp
from jax import lax
from jax.experimental import pallas as pl
from jax.experimental.pallas import tpu as pltpu
from jax.experimental.pallas import tpu_sc as plsc

N = 10000
E = 320000
D = 128
WIN = 128
NC = 2
NS = 16
WPT = 80
CW = 16
NCHUNK = 5
NWINP = NC * NS * WPT
EPAD = NWINP * WIN - E
ACC_ROWS = 10240
RPT = ACC_ROWS // NS

_mesh = plsc.VectorSubcoreMesh(core_axis_name="c", subcore_axis_name="s")


def _zero_fill(buf, rows, width):
    zero = jnp.zeros((16,), jnp.float32)

    def row(i, _):
        for j in range(width // 16):
            buf[i, pl.ds(j * 16, 16)] = zero
        return 0

    lax.fori_loop(0, rows, row, 0)


def _zero_acc(acc, zbuf, s, width):
    _zero_fill(zbuf, 128, width)
    for k in range(RPT // 128):
        pltpu.sync_copy(zbuf, acc.at[pl.ds(s * RPT + k * 128, 128)])


def _writeback(acc, outp, c, s):
    for k in range(RPT // 128):
        pltpu.sync_copy(acc.at[pl.ds(s * RPT + k * 128, 128)],
                        outp.at[c, pl.ds(s * RPT + k * 128, 128)])


def _deg_body(dst2d, outp, acc, idxbuf, onesbuf, dsem):
    c = lax.axis_index("c")
    s = lax.axis_index("s")
    base = (c * NS + s) * WPT

    _zero_acc(acc, onesbuf, s, 16)
    pltpu.sync_copy(dst2d.at[pl.ds(base, WPT)], idxbuf)

    one = jnp.ones((16,), jnp.float32)

    def fill(i, _):
        onesbuf[i, :] = one
        return 0

    lax.fori_loop(0, 128, fill, 0)
    plsc.subcore_barrier()

    def win(j, _):
        pltpu.async_copy(onesbuf, acc.at[idxbuf.at[j]], dsem, add=True)

        @pl.when(j >= 8)
        def _():
            pltpu.make_async_copy(onesbuf, acc.at[idxbuf.at[j]], dsem).wait()

        return 0

    lax.fori_loop(0, WPT, win, 0)
    for _ in range(8):
        pltpu.make_async_copy(onesbuf, acc.at[idxbuf.at[0]], dsem).wait()
    plsc.subcore_barrier()
    _writeback(acc, outp, c, s)


_deg_call = pl.kernel(
    _deg_body,
    out_type=jax.ShapeDtypeStruct((NC, ACC_ROWS, 16), jnp.float32),
    mesh=_mesh,
    scratch_types=[
        pltpu.VMEM_SHARED((ACC_ROWS, 16), jnp.float32),
        pltpu.VMEM((WPT, WIN), jnp.int32),
        pltpu.VMEM((128, 16), jnp.float32),
        pltpu.SemaphoreType.DMA,
    ],
)


def _spmm_body(xp, src2d, dst2d, outp, acc, sbuf, dbuf, rbuf0, rbuf1,
               gsem0, gsem1, ssem0, ssem1, isem):
    c = lax.axis_index("c")
    s = lax.axis_index("s")
    base = (c * NS + s) * WPT

    _zero_acc(acc, rbuf0, s, D)
    plsc.subcore_barrier()

    pltpu.sync_copy(src2d.at[pl.ds(base, CW)], sbuf.at[0])
    pltpu.sync_copy(dst2d.at[pl.ds(base, CW)], dbuf.at[0])
    pltpu.async_copy(xp.at[sbuf.at[0, 0]], rbuf0, gsem0)

    def chunk(ch, _):
        par = ch % 2
        nxt = 1 - par

        @pl.when(ch + 1 < NCHUNK)
        def _():
            pltpu.async_copy(src2d.at[pl.ds(base + (ch + 1) * CW, CW)],
                             sbuf.at[nxt], isem)
            pltpu.async_copy(dst2d.at[pl.ds(base + (ch + 1) * CW, CW)],
                             dbuf.at[nxt], isem)

        def step(t, _):
            j0 = 2 * t
            j1 = j0 + 1
            pltpu.make_async_copy(xp.at[sbuf.at[par, j0]], rbuf0, gsem0).wait()
            pltpu.async_copy(rbuf0, acc.at[dbuf.at[par, j0]], ssem0, add=True)

            @pl.when(t > 0)
            def _():
                pltpu.make_async_copy(
                    rbuf1, acc.at[dbuf.at[par, j1 - 2]], ssem1).wait()

            pltpu.async_copy(xp.at[sbuf.at[par, j1]], rbuf1, gsem1)
            pltpu.make_async_copy(xp.at[sbuf.at[par, j1]], rbuf1, gsem1).wait()
            pltpu.async_copy(rbuf1, acc.at[dbuf.at[par, j1]], ssem1, add=True)
            pltpu.make_async_copy(rbuf0, acc.at[dbuf.at[par, j0]], ssem0).wait()

            @pl.when(j0 + 2 < CW)
            def _():
                pltpu.async_copy(xp.at[sbuf.at[par, j0 + 2]], rbuf0, gsem0)

            return 0

        lax.fori_loop(0, CW // 2, step, 0)
        pltpu.make_async_copy(rbuf1, acc.at[dbuf.at[par, CW - 1]], ssem1).wait()

        @pl.when(ch + 1 < NCHUNK)
        def _():
            pltpu.make_async_copy(src2d.at[pl.ds(base + (ch + 1) * CW, CW)],
                                  sbuf.at[nxt], isem).wait()
            pltpu.make_async_copy(dst2d.at[pl.ds(base + (ch + 1) * CW, CW)],
                                  dbuf.at[nxt], isem).wait()
            pltpu.async_copy(xp.at[sbuf.at[nxt, 0]], rbuf0, gsem0)

        return 0

    lax.fori_loop(0, NCHUNK, chunk, 0)
    plsc.subcore_barrier()
    _writeback(acc, outp, c, s)


_spmm_call = pl.kernel(
    _spmm_body,
    out_type=jax.ShapeDtypeStruct((NC, ACC_ROWS, D), jnp.float32),
    mesh=_mesh,
    scratch_types=[
        pltpu.VMEM_SHARED((ACC_ROWS, D), jnp.float32),
        pltpu.VMEM((2, CW, WIN), jnp.int32),
        pltpu.VMEM((2, CW, WIN), jnp.int32),
        pltpu.VMEM((WIN, D), jnp.float32),
        pltpu.VMEM((WIN, D), jnp.float32),
        pltpu.SemaphoreType.DMA,
        pltpu.SemaphoreType.DMA,
        pltpu.SemaphoreType.DMA,
        pltpu.SemaphoreType.DMA,
        pltpu.SemaphoreType.DMA,
    ],
)


BR = 2000
_GRID = (N // BR,)


def _dinv_of(degp):
    return lax.rsqrt(degp[0, :, 0:1] + degp[1, :, 0:1] + 1.0)


def _scale_body(degp, x, o):
    o[...] = x[...] * _dinv_of(degp)


def _layer1_body(degp, s0p, xp, w1, b1, o):
    dinv = _dinv_of(degp)
    p = (s0p[0] + s0p[1] + xp[...]) * dinv
    h = jnp.dot(p, w1[...], preferred_element_type=jnp.float32) + b1[...]
    o[...] = jnp.maximum(h, 0.0) * dinv


def _layer2_body(degp, s1p, h1p, w2, b2, wcat, o):
    dinv = _dinv_of(degp)
    p = (s1p[0] + s1p[1] + h1p[...]) * dinv
    h2 = jnp.dot(p, w2[...], preferred_element_type=jnp.float32) + b2[...]
    h2 = jnp.maximum(h2, 0.0)
    o[...] = jnp.dot(h2, wcat[...], preferred_element_type=jnp.float32) * dinv


def _out_body(degp, s2p, gp, bmu, bls, mu, ls):
    dinv = _dinv_of(degp)
    p = (s2p[0] + s2p[1] + gp[...]) * dinv
    mu[...] = p[:, :64] + bmu[...]
    ls[...] = p[:, 64:] + bls[...]


def _rows(block_w):
    return pl.BlockSpec((BR, block_w), lambda i: (i, 0))


def _parts(block_w):
    return pl.BlockSpec((NC, BR, block_w), lambda i: (0, i, 0))


def _full(shape):
    nd = len(shape)
    return pl.BlockSpec(shape, lambda i: (0,) * nd)


_degp_spec = _parts(16)

_scale = pl.pallas_call(
    _scale_body,
    grid=_GRID,
    in_specs=[_degp_spec, _rows(D)],
    out_specs=_rows(D),
    out_shape=jax.ShapeDtypeStruct((N, D), jnp.float32),
)

_layer1 = pl.pallas_call(
    _layer1_body,
    grid=_GRID,
    in_specs=[_degp_spec, _parts(D), _rows(D), _full((D, D)), _full((1, D))],
    out_specs=_rows(D),
    out_shape=jax.ShapeDtypeStruct((N, D), jnp.float32),
)

_layer2 = pl.pallas_call(
    _layer2_body,
    grid=_GRID,
    in_specs=[_degp_spec, _parts(D), _rows(D), _full((D, 256)),
              _full((1, 256)), _full((256, D))],
    out_specs=_rows(D),
    out_shape=jax.ShapeDtypeStruct((N, D), jnp.float32),
)

_outs = pl.pallas_call(
    _out_body,
    grid=_GRID,
    in_specs=[_degp_spec, _parts(D), _rows(D), _full((1, 64)), _full((1, 64))],
    out_specs=[_rows(64), _rows(64)],
    out_shape=[jax.ShapeDtypeStruct((N, 64), jnp.float32),
               jax.ShapeDtypeStruct((N, 64), jnp.float32)],
)


def kernel(x, edge_index, W1, b1, W2, b2, Wmu, bmu, Wls, bls):
    ar = jnp.arange(EPAD, dtype=jnp.int32)
    pad_src = ar % 4096
    pad_dst = N + (ar % (ACC_ROWS - N))
    src2d = jnp.concatenate([edge_index[0], pad_src]).reshape(NWINP, WIN)
    dst2d = jnp.concatenate([edge_index[1], pad_dst]).reshape(NWINP, WIN)
    wcat = jnp.concatenate([Wmu, Wls], axis=1)
    b1r = b1.reshape(1, D)
    b2r = b2.reshape(1, 256)
    bmur = bmu.reshape(1, 64)
    blsr = bls.reshape(1, 64)

    degp = _deg_call(dst2d)
    xp = _scale(degp, x)
    s0p = _spmm_call(xp, src2d, dst2d)
    h1p = _layer1(degp, s0p, xp, W1, b1r)
    s1p = _spmm_call(h1p, src2d, dst2d)
    gp = _layer2(degp, s1p, h1p, W2, b2r, wcat)
    s2p = _spmm_call(gp, src2d, dst2d)
    mu, ls = _outs(degp, s2p, gp, bmur, blsr)
    return (mu, ls)
